# baseline (speedup 1.0000x reference)
import functools

import jax
import jax.numpy as jnp
from jax import lax
from jax.experimental import pallas as pl
from jax.experimental.pallas import tpu as pltpu
from jax.experimental.pallas import tpu_sc as plsc

N = 10000
NP = 10240
E = 320000
EP = 327680
H = 128
G = 64
EDGE_DIM = 16

NBLK = 2048
EBLK = 4096
NTILES = 32
EPT = EP // NTILES
KE = 64
NITER = EPT // KE
ROWS_PT = NP // 16
SROWS = 10008
SSLICE = 632
_SCHUNKS = [(0, 64), (64, 64), (128, 64), (192, 64), (256, 64),
            (320, 64), (384, 64), (448, 64), (512, 64), (576, 56)]

_F32 = jnp.float32


def _sig(x):
    return 1.0 / (1.0 + jnp.exp(-x))


def _layernorm(x, g, b):
    m = jnp.mean(x, axis=-1, keepdims=True)
    v = jnp.mean((x - m) ** 2, axis=-1, keepdims=True)
    return (x - m) * lax.rsqrt(v + 1e-5) * g + b


def _embed_body(x_ref, w1, b1, g1, bb1, w2, b2, wa, wb, o_ref, a_ref, b_ref):
    h = jnp.dot(x_ref[...], w1[...], preferred_element_type=_F32) + b1[...]
    h = h * _sig(h)
    h = _layernorm(h, g1[...], bb1[...])
    h = jnp.dot(h, w2[...], preferred_element_type=_F32) + b2[...]
    o_ref[...] = h
    a_ref[...] = jnp.dot(h, wa[...], preferred_element_type=_F32)
    b_ref[...] = jnp.dot(h, wb[...], preferred_element_type=_F32)


def _ec_body(ea_ref, w, b, o_ref):
    o_ref[...] = jnp.dot(ea_ref[...], w[...], preferred_element_type=_F32) + b[...]


def _upd_val(s0, s1, inv, msk, h, mw2, mb2, uw1a, uw1b, ub1, uw2, ub2, g, bb):
    agg = (s0 + s1) * inv
    aggm = jnp.dot(agg, mw2, preferred_element_type=_F32) + mb2 * msk
    u = (jnp.dot(h, uw1a, preferred_element_type=_F32)
         + jnp.dot(aggm, uw1b, preferred_element_type=_F32) + ub1)
    u = u * _sig(u)
    u = jnp.dot(u, uw2, preferred_element_type=_F32) + ub2
    return _layernorm(u + h, g, bb)


def _upd_body(s0, s1, inv, msk, h_ref, mw2, mb2, uw1a, uw1b, ub1, uw2, ub2,
              g, bb, o_ref):
    o_ref[...] = _upd_val(
        s0[...], s1[...], inv[...], msk[...], h_ref[...], mw2[...], mb2[...],
        uw1a[...], uw1b[...], ub1[...], uw2[...], ub2[...], g[...], bb[...])


def _updp_body(s0, s1, inv, msk, h_ref, mw2, mb2, uw1a, uw1b, ub1, uw2, ub2,
               g, bb, wa, wb, o_ref, a_ref, b_ref):
    hn = _upd_val(
        s0[...], s1[...], inv[...], msk[...], h_ref[...], mw2[...], mb2[...],
        uw1a[...], uw1b[...], ub1[...], uw2[...], ub2[...], g[...], bb[...])
    o_ref[...] = hn
    a_ref[...] = jnp.dot(hn, wa[...], preferred_element_type=_F32)
    b_ref[...] = jnp.dot(hn, wb[...], preferred_element_type=_F32)


def _upd0p_body(s0, s1, c0, c1, h_ref, mw2, mb2, uw1a, uw1b, ub1, uw2, ub2,
                g, bb, wa, wb, o_ref, inv_ref, msk_ref, a_ref, b_ref):
    cnt = c0[...] + c1[...]
    inv = 1.0 / jnp.maximum(cnt, 1.0)
    msk = (cnt > 0.0).astype(_F32)
    inv_ref[...] = inv
    msk_ref[...] = msk
    hn = _upd_val(
        s0[...], s1[...], inv, msk, h_ref[...], mw2[...], mb2[...],
        uw1a[...], uw1b[...], ub1[...], uw2[...], ub2[...], g[...], bb[...])
    o_ref[...] = hn
    a_ref[...] = jnp.dot(hn, wa[...], preferred_element_type=_F32)
    b_ref[...] = jnp.dot(hn, wb[...], preferred_element_type=_F32)


def _pool_body(h_ref, bt_ref, o_ref, acc, cac):
    i = pl.program_id(0)

    @pl.when(i == 0)
    def _():
        acc[...] = jnp.zeros_like(acc)
        cac[...] = jnp.zeros_like(cac)

    m = (bt_ref[...] == lax.broadcasted_iota(jnp.int32, (NBLK, G), 1)
         ).astype(_F32)
    h = h_ref[...]
    dn = (((0,), (0,)), ((), ()))
    acc[...] += lax.dot_general(m, h, dn, preferred_element_type=_F32)
    cac[...] += lax.dot_general(m, jnp.ones_like(h), dn,
                                preferred_element_type=_F32)

    @pl.when(i == pl.num_programs(0) - 1)
    def _():
        o_ref[...] = acc[...] / jnp.maximum(cac[...], 1.0)


def _heada_body(x_ref, w1, b1, g1, bb1, w2, b2, g2, bb2, o_ref):
    t = jnp.dot(x_ref[...], w1[...], preferred_element_type=_F32) + b1[...]
    t = t * _sig(t)
    t = _layernorm(t, g1[...], bb1[...])
    t = jnp.dot(t, w2[...], preferred_element_type=_F32) + b2[...]
    t = t * _sig(t)
    o_ref[...] = _layernorm(t, g2[...], bb2[...])


def _headb_body(x_ref, w3, b3, o_ref):
    o_ref[...] = jnp.dot(x_ref[...], w3[...], preferred_element_type=_F32) + b3[...]


def _full(shape):
    return pl.BlockSpec(shape, lambda *_: tuple(0 for _ in shape))


def _rows(shape):
    return pl.BlockSpec(shape, lambda i: (i,) + tuple(0 for _ in shape[1:]))


def _embed(x, w1, b1, g1, bb1, w2, b2, wa, wb):
    return pl.pallas_call(
        _embed_body,
        grid=(NP // NBLK,),
        in_specs=[_rows((NBLK, H)), _full((H, H)), _full((1, H)),
                  _full((1, H)), _full((1, H)), _full((H, H)), _full((1, H)),
                  _full((H, H)), _full((H, H))],
        out_specs=[_rows((NBLK, H)), _rows((NBLK, H)), _rows((NBLK, H))],
        out_shape=[jax.ShapeDtypeStruct((NP, H), _F32),
                   jax.ShapeDtypeStruct((NP, H), _F32),
                   jax.ShapeDtypeStruct((NP, H), _F32)],
    )(x, w1, b1, g1, bb1, w2, b2, wa, wb)


def _ec(ea, w, b):
    return pl.pallas_call(
        _ec_body,
        grid=(EP // EBLK,),
        in_specs=[_rows((EBLK, EDGE_DIM)), _full((EDGE_DIM, H)), _full((1, H))],
        out_specs=_rows((EBLK, H)),
        out_shape=jax.ShapeDtypeStruct((EP, H), _F32),
    )(ea, w, b)


def _upd(s0, s1, inv, msk, h, mw2, mb2, uw1a, uw1b, ub1, uw2, ub2, g, bb):
    return pl.pallas_call(
        _upd_body,
        grid=(NP // NBLK,),
        in_specs=[_rows((NBLK, H)), _rows((NBLK, H)), _rows((NBLK, 1)),
                  _rows((NBLK, 1)), _rows((NBLK, H)),
                  _full((H, H)), _full((1, H)),
                  _full((H, 2 * H)), _full((H, 2 * H)), _full((1, 2 * H)),
                  _full((2 * H, H)), _full((1, H)),
                  _full((1, H)), _full((1, H))],
        out_specs=_rows((NBLK, H)),
        out_shape=jax.ShapeDtypeStruct((NP, H), _F32),
    )(s0, s1, inv, msk, h, mw2, mb2, uw1a, uw1b, ub1, uw2, ub2, g, bb)


def _updp(s0, s1, inv, msk, h, mw2, mb2, uw1a, uw1b, ub1, uw2, ub2, g, bb,
          wa, wb):
    return pl.pallas_call(
        _updp_body,
        grid=(NP // NBLK,),
        in_specs=[_rows((NBLK, H)), _rows((NBLK, H)), _rows((NBLK, 1)),
                  _rows((NBLK, 1)), _rows((NBLK, H)),
                  _full((H, H)), _full((1, H)),
                  _full((H, 2 * H)), _full((H, 2 * H)), _full((1, 2 * H)),
                  _full((2 * H, H)), _full((1, H)),
                  _full((1, H)), _full((1, H)),
                  _full((H, H)), _full((H, H))],
        out_specs=[_rows((NBLK, H)), _rows((NBLK, H)), _rows((NBLK, H))],
        out_shape=[jax.ShapeDtypeStruct((NP, H), _F32),
                   jax.ShapeDtypeStruct((NP, H), _F32),
                   jax.ShapeDtypeStruct((NP, H), _F32)],
    )(s0, s1, inv, msk, h, mw2, mb2, uw1a, uw1b, ub1, uw2, ub2, g, bb, wa, wb)


def _upd0p(s0, s1, c0, c1, h, mw2, mb2, uw1a, uw1b, ub1, uw2, ub2, g, bb,
           wa, wb):
    return pl.pallas_call(
        _upd0p_body,
        grid=(NP // NBLK,),
        in_specs=[_rows((NBLK, H)), _rows((NBLK, H)), _rows((NBLK, 1)),
                  _rows((NBLK, 1)), _rows((NBLK, H)),
                  _full((H, H)), _full((1, H)),
                  _full((H, 2 * H)), _full((H, 2 * H)), _full((1, 2 * H)),
                  _full((2 * H, H)), _full((1, H)),
                  _full((1, H)), _full((1, H)),
                  _full((H, H)), _full((H, H))],
        out_specs=[_rows((NBLK, H)), _rows((NBLK, 1)), _rows((NBLK, 1)),
                   _rows((NBLK, H)), _rows((NBLK, H))],
        out_shape=[jax.ShapeDtypeStruct((NP, H), _F32),
                   jax.ShapeDtypeStruct((NP, 1), _F32),
                   jax.ShapeDtypeStruct((NP, 1), _F32),
                   jax.ShapeDtypeStruct((NP, H), _F32),
                   jax.ShapeDtypeStruct((NP, H), _F32)],
    )(s0, s1, c0, c1, h, mw2, mb2, uw1a, uw1b, ub1, uw2, ub2, g, bb, wa, wb)


def _pool(h, bt):
    return pl.pallas_call(
        _pool_body,
        grid=(NP // NBLK,),
        in_specs=[_rows((NBLK, H)), _rows((NBLK, 1))],
        out_specs=_full((G, H)),
        out_shape=jax.ShapeDtypeStruct((G, H), _F32),
        scratch_shapes=[pltpu.VMEM((G, H), _F32), pltpu.VMEM((G, H), _F32)],
    )(h, bt)


def _head_a(x, w1, b1, g1, bb1, w2, b2, g2, bb2):
    return pl.pallas_call(
        _heada_body,
        in_specs=[_full((G, H)), _full((H, 2 * H)), _full((1, 2 * H)),
                  _full((1, 2 * H)), _full((1, 2 * H)),
                  _full((2 * H, 4 * H)), _full((1, 4 * H)),
                  _full((1, 4 * H)), _full((1, 4 * H))],
        out_specs=_full((G, 4 * H)),
        out_shape=jax.ShapeDtypeStruct((G, 4 * H), _F32),
    )(x, w1, b1, g1, bb1, w2, b2, g2, bb2)


def _head_b(x, w3, b3):
    cblk = 3840
    nout = 120 * 256
    return pl.pallas_call(
        _headb_body,
        grid=(nout // cblk,),
        in_specs=[_full((G, 4 * H)),
                  pl.BlockSpec((4 * H, cblk), lambda i: (0, i)),
                  pl.BlockSpec((1, cblk), lambda i: (0, i))],
        out_specs=pl.BlockSpec((G, cblk), lambda i: (0, i)),
        out_shape=jax.ShapeDtypeStruct((G, nout), _F32),
    )(x, w3, b3)


def _make_edge_kernel(with_cnt):
    mesh = plsc.VectorSubcoreMesh(core_axis_name="c", subcore_axis_name="s")
    out_type = [jax.ShapeDtypeStruct((2, SROWS, H), _F32)]
    if with_cnt:
        out_type.append(jax.ShapeDtypeStruct((2, NP), _F32))
    scratch = [
        pltpu.VMEM((KE,), jnp.int32),
        pltpu.VMEM((KE,), jnp.int32),
        pltpu.VMEM((KE, H), _F32),
        pltpu.VMEM((KE, H), _F32),
        pltpu.VMEM((KE, H), _F32),
        pltpu.VMEM((KE,), jnp.int32),
        pltpu.VMEM((KE,), jnp.int32),
        pltpu.VMEM((KE, H), _F32),
        pltpu.VMEM((KE, H), _F32),
        pltpu.VMEM((KE, H), _F32),
        pltpu.VMEM((128,), _F32),
        pltpu.VMEM_SHARED((SROWS, H), _F32),
        pltpu.VMEM_SHARED((NP,), _F32),
        pltpu.SemaphoreType.DMA,
        pltpu.SemaphoreType.DMA,
        pltpu.SemaphoreType.DMA,
        pltpu.SemaphoreType.DMA,
        pltpu.SemaphoreType.DMA,
        pltpu.SemaphoreType.DMA,
        pltpu.SemaphoreType.DMA,
        pltpu.SemaphoreType.DMA,
    ]

    def body(a_hbm, b_hbm, ec_hbm, dst_hbm, src_hbm, *rest):
        if with_cnt:
            s_out, cnt_out = rest[0], rest[1]
            rest = rest[2:]
        else:
            s_out, cnt_out = rest[0], None
            rest = rest[1:]
        (dv0, sv0, ab0, bb0, eb0,
         dv1, sv1, ab1, bb1, eb1,
         onesv, s_sh, c_sh,
         semA0, semB0, semE0, semA1, semB1, semE1, semS0, semS1) = rest
        bufs = ((dv0, sv0, ab0, bb0, eb0, semA0, semB0, semE0, semS0),
                (dv1, sv1, ab1, bb1, eb1, semA1, semB1, semE1, semS1))
        c = lax.axis_index("c")
        s = lax.axis_index("s")
        wid = c * 16 + s
        r0 = jnp.minimum(s * SSLICE, SROWS - SSLICE)

        def zrow(r, _):
            for cc in range(H // 16):
                ab0[r, pl.ds(cc * 16, 16)] = jnp.zeros((16,), _F32)
            return 0
        lax.fori_loop(0, KE, zrow, 0)
        for off, sz in _SCHUNKS:
            pltpu.sync_copy(ab0.at[pl.ds(0, sz)],
                            s_sh.at[pl.ds(r0 + off, sz)])
        if with_cnt:
            def zv(j, _):
                onesv[pl.ds(j * 16, 16)] = jnp.zeros((16,), _F32)
                return 0
            lax.fori_loop(0, 8, zv, 0)
            for k in range(ROWS_PT // 128):
                pltpu.sync_copy(
                    onesv, c_sh.at[pl.ds(s * ROWS_PT + k * 128, 128)])

            def ov(j, _):
                onesv[pl.ds(j * 16, 16)] = jnp.full((16,), 1.0, _F32)
                return 0
            lax.fori_loop(0, 8, ov, 0)
        plsc.subcore_barrier()

        ebase = wid * EPT

        def prefetch(i, bset):
            dv, sv, ab, bb, eb, sA, sB, sE, sS = bset
            off = jnp.minimum(ebase + i * KE, EP - KE)
            pltpu.sync_copy(dst_hbm.at[pl.ds(off, KE)], dv)
            pltpu.sync_copy(src_hbm.at[pl.ds(off, KE)], sv)
            pltpu.async_copy(a_hbm.at[dv], ab, sA)
            pltpu.async_copy(b_hbm.at[sv], bb, sB)
            pltpu.async_copy(ec_hbm.at[pl.ds(off, KE)], eb, sE)

        def phase(i, cur, nxt):
            ndv, _, nab, _, _, _, _, _, nsS = nxt

            @pl.when(i >= 1)
            def _():
                pltpu.make_async_copy(nab, s_sh.at[ndv], nsS).wait()
            prefetch(i + 1, nxt)
            dv, sv, ab, bb, eb, sA, sB, sE, sS = cur
            pltpu.make_async_copy(a_hbm.at[dv], ab, sA).wait()
            pltpu.make_async_copy(b_hbm.at[sv], bb, sB).wait()
            pltpu.make_async_copy(ec_hbm.at[pl.ds(0, KE)], eb, sE).wait()

            @plsc.parallel_loop(0, KE, 1, unroll=2)
            def _(r2):
                for cc in range(H // 16):
                    sl = pl.ds(cc * 16, 16)
                    xv = ab[r2, sl] + bb[r2, sl] + eb[r2, sl]
                    ab[r2, sl] = xv / (1.0 + jnp.exp(-xv))

            pltpu.async_copy(ab, s_sh.at[dv], sS, add=True)
            if with_cnt:
                pltpu.sync_copy(onesv.at[pl.ds(0, KE)], c_sh.at[dv], add=True)

        prefetch(0, bufs[0])

        def gstep(g, _):
            phase(2 * g, bufs[0], bufs[1])
            phase(2 * g + 1, bufs[1], bufs[0])
            return 0
        lax.fori_loop(0, NITER // 2, gstep, 0)
        pltpu.make_async_copy(a_hbm.at[dv0], ab0, semA0).wait()
        pltpu.make_async_copy(b_hbm.at[sv0], bb0, semB0).wait()
        pltpu.make_async_copy(ec_hbm.at[pl.ds(0, KE)], eb0, semE0).wait()
        pltpu.make_async_copy(ab1, s_sh.at[dv1], semS1).wait()
        plsc.subcore_barrier()

        for off, sz in _SCHUNKS:
            rr = r0 + off
            pltpu.sync_copy(s_sh.at[pl.ds(rr, sz)], ab0.at[pl.ds(0, sz)])
            pltpu.sync_copy(ab0.at[pl.ds(0, sz)], s_out.at[c, pl.ds(rr, sz)])
        if with_cnt:
            for k in range(ROWS_PT // 128):
                rr = s * ROWS_PT + k * 128
                pltpu.sync_copy(c_sh.at[pl.ds(rr, 128)], onesv)
                pltpu.sync_copy(onesv, cnt_out.at[c, pl.ds(rr, 128)])

    return functools.partial(
        pl.kernel, mesh=mesh, out_type=out_type, scratch_types=scratch)(body)


_edge_cnt = _make_edge_kernel(True)
_edge = _make_edge_kernel(False)


def kernel(x, edge_index, edge_attr, batch, params):
    p = params
    x_p = jnp.pad(x, ((0, NP - N), (0, 0)))
    src = edge_index[0].astype(jnp.int32)
    dst = edge_index[1].astype(jnp.int32)
    pe = EP - E
    dst_p = jnp.concatenate([dst, jnp.full((pe,), N, jnp.int32)])
    src_p = jnp.concatenate([src, jnp.zeros((pe,), jnp.int32)])
    ea_p = jnp.pad(edge_attr, ((0, pe), (0, 0)))
    bt_p = jnp.concatenate(
        [batch.astype(jnp.int32), jnp.full((NP - N,), G, jnp.int32)]
    ).reshape(NP, 1)

    def r1(v):
        return v.reshape(1, -1)

    layers = p['layers']

    def msg_wa(lp):
        return lp['msg_w1'][:H]

    def msg_wb(lp):
        return lp['msg_w1'][H:2 * H]

    h, a, b = _embed(x_p, p['emb_w1'], r1(p['emb_b1']), r1(p['emb_ln_g']),
                     r1(p['emb_ln_b']), p['emb_w2'], r1(p['emb_b2']),
                     msg_wa(layers[0]), msg_wb(layers[0]))

    inv = msk = None
    for li, lp in enumerate(layers):
        wc = lp['msg_w1'][2 * H:]
        ec = _ec(ea_p, wc, r1(lp['msg_b1']))
        uw1a = lp['upd_w1'][:H]
        uw1b = lp['upd_w1'][H:]
        uargs = (lp['msg_w2'], r1(lp['msg_b2']), uw1a, uw1b,
                 r1(lp['upd_b1']), lp['upd_w2'], r1(lp['upd_b2']),
                 r1(lp['ln_g']), r1(lp['ln_b']))
        if li == 0:
            s, cnt = _edge_cnt(a, b, ec, dst_p, src_p)
            s = jnp.pad(s, ((0, 0), (0, NP - SROWS), (0, 0)))
            h, inv, msk, a, b = _upd0p(
                s[0], s[1], cnt[0].reshape(NP, 1), cnt[1].reshape(NP, 1), h,
                *uargs, msg_wa(layers[1]), msg_wb(layers[1]))
        else:
            s = _edge(a, b, ec, dst_p, src_p)
            if isinstance(s, (list, tuple)):
                s = s[0]
            s = jnp.pad(s, ((0, 0), (0, NP - SROWS), (0, 0)))
            if li + 1 < len(layers):
                h, a, b = _updp(s[0], s[1], inv, msk, h, *uargs,
                                msg_wa(layers[li + 1]), msg_wb(layers[li + 1]))
            else:
                h = _upd(s[0], s[1], inv, msk, h, *uargs)

    gemb = _pool(h, bt_p)
    o2 = _head_a(gemb, p['head_w1'], r1(p['head_b1']), r1(p['head_ln1_g']),
                 r1(p['head_ln1_b']), p['head_w2'], r1(p['head_b2']),
                 r1(p['head_ln2_g']), r1(p['head_ln2_b']))
    out = _head_b(o2, p['head_w3'], r1(p['head_b3']))
    return out.reshape(G, 120, 256)


# speedup vs baseline: 1.0102x; 1.0102x over previous
import functools

import jax
import jax.numpy as jnp
from jax import lax
from jax.experimental import pallas as pl
from jax.experimental.pallas import tpu as pltpu
from jax.experimental.pallas import tpu_sc as plsc

N = 10000
NP = 10240
E = 320000
EP = 327680
H = 128
G = 64
EDGE_DIM = 16

NBLK = 2048
EBLK = 4096
NTILES = 32
EPT = EP // NTILES
KE = 64
NITER = EPT // KE
ROWS_PT = NP // 16
SROWS = 10008
SSLICE = 632
_SCHUNKS = [(0, 64), (64, 64), (128, 64), (192, 64), (256, 64),
            (320, 64), (384, 64), (448, 64), (512, 64), (576, 56)]

_F32 = jnp.float32


def _sig(x):
    return 1.0 / (1.0 + jnp.exp(-x))


def _layernorm(x, g, b):
    m = jnp.mean(x, axis=-1, keepdims=True)
    v = jnp.mean((x - m) ** 2, axis=-1, keepdims=True)
    return (x - m) * lax.rsqrt(v + 1e-5) * g + b


def _embed_body(x_ref, w1, b1, g1, bb1, w2, b2, wa, wb, o_ref, a_ref, b_ref):
    h = jnp.dot(x_ref[...], w1[...], preferred_element_type=_F32) + b1[...]
    h = h * _sig(h)
    h = _layernorm(h, g1[...], bb1[...])
    h = jnp.dot(h, w2[...], preferred_element_type=_F32) + b2[...]
    o_ref[...] = h
    a_ref[...] = jnp.dot(h, wa[...], preferred_element_type=_F32)
    b_ref[...] = jnp.dot(h, wb[...], preferred_element_type=_F32)


def _ec_body(ea_ref, w, b, o_ref):
    o_ref[...] = jnp.dot(ea_ref[...], w[...], preferred_element_type=_F32) + b[...]


def _upd_val(s0, s1, inv, msk, h, mw2, mb2, uw1a, uw1b, ub1, uw2, ub2, g, bb):
    agg = (s0 + s1) * inv
    aggm = jnp.dot(agg, mw2, preferred_element_type=_F32) + mb2 * msk
    u = (jnp.dot(h, uw1a, preferred_element_type=_F32)
         + jnp.dot(aggm, uw1b, preferred_element_type=_F32) + ub1)
    u = u * _sig(u)
    u = jnp.dot(u, uw2, preferred_element_type=_F32) + ub2
    return _layernorm(u + h, g, bb)


def _upd_body(s0, s1, inv, msk, h_ref, mw2, mb2, uw1a, uw1b, ub1, uw2, ub2,
              g, bb, o_ref):
    o_ref[...] = _upd_val(
        s0[...], s1[...], inv[...], msk[...], h_ref[...], mw2[...], mb2[...],
        uw1a[...], uw1b[...], ub1[...], uw2[...], ub2[...], g[...], bb[...])


def _updp_body(s0, s1, inv, msk, h_ref, mw2, mb2, uw1a, uw1b, ub1, uw2, ub2,
               g, bb, wa, wb, o_ref, a_ref, b_ref):
    hn = _upd_val(
        s0[...], s1[...], inv[...], msk[...], h_ref[...], mw2[...], mb2[...],
        uw1a[...], uw1b[...], ub1[...], uw2[...], ub2[...], g[...], bb[...])
    o_ref[...] = hn
    a_ref[...] = jnp.dot(hn, wa[...], preferred_element_type=_F32)
    b_ref[...] = jnp.dot(hn, wb[...], preferred_element_type=_F32)


def _upd0p_body(s0, s1, c0, c1, h_ref, mw2, mb2, uw1a, uw1b, ub1, uw2, ub2,
                g, bb, wa, wb, o_ref, inv_ref, msk_ref, a_ref, b_ref):
    cnt = c0[...] + c1[...]
    inv = 1.0 / jnp.maximum(cnt, 1.0)
    msk = (cnt > 0.0).astype(_F32)
    inv_ref[...] = inv
    msk_ref[...] = msk
    hn = _upd_val(
        s0[...], s1[...], inv, msk, h_ref[...], mw2[...], mb2[...],
        uw1a[...], uw1b[...], ub1[...], uw2[...], ub2[...], g[...], bb[...])
    o_ref[...] = hn
    a_ref[...] = jnp.dot(hn, wa[...], preferred_element_type=_F32)
    b_ref[...] = jnp.dot(hn, wb[...], preferred_element_type=_F32)


def _pool_body(h_ref, bt_ref, o_ref, acc, cac):
    i = pl.program_id(0)

    @pl.when(i == 0)
    def _():
        acc[...] = jnp.zeros_like(acc)
        cac[...] = jnp.zeros_like(cac)

    m = (bt_ref[...] == lax.broadcasted_iota(jnp.int32, (NBLK, G), 1)
         ).astype(_F32)
    h = h_ref[...]
    dn = (((0,), (0,)), ((), ()))
    acc[...] += lax.dot_general(m, h, dn, preferred_element_type=_F32)
    cac[...] += lax.dot_general(m, jnp.ones_like(h), dn,
                                preferred_element_type=_F32)

    @pl.when(i == pl.num_programs(0) - 1)
    def _():
        o_ref[...] = acc[...] / jnp.maximum(cac[...], 1.0)


def _heada_body(x_ref, w1, b1, g1, bb1, w2, b2, g2, bb2, o_ref):
    t = jnp.dot(x_ref[...], w1[...], preferred_element_type=_F32) + b1[...]
    t = t * _sig(t)
    t = _layernorm(t, g1[...], bb1[...])
    t = jnp.dot(t, w2[...], preferred_element_type=_F32) + b2[...]
    t = t * _sig(t)
    o_ref[...] = _layernorm(t, g2[...], bb2[...])


def _headb_body(x_ref, w3, b3, o_ref):
    o_ref[...] = jnp.dot(x_ref[...], w3[...], preferred_element_type=_F32) + b3[...]


def _full(shape):
    return pl.BlockSpec(shape, lambda *_: tuple(0 for _ in shape))


def _rows(shape):
    return pl.BlockSpec(shape, lambda i: (i,) + tuple(0 for _ in shape[1:]))


def _embed(x, w1, b1, g1, bb1, w2, b2, wa, wb):
    return pl.pallas_call(
        _embed_body,
        grid=(NP // NBLK,),
        in_specs=[_rows((NBLK, H)), _full((H, H)), _full((1, H)),
                  _full((1, H)), _full((1, H)), _full((H, H)), _full((1, H)),
                  _full((H, H)), _full((H, H))],
        out_specs=[_rows((NBLK, H)), _rows((NBLK, H)), _rows((NBLK, H))],
        out_shape=[jax.ShapeDtypeStruct((NP, H), _F32),
                   jax.ShapeDtypeStruct((NP, H), _F32),
                   jax.ShapeDtypeStruct((NP, H), _F32)],
    )(x, w1, b1, g1, bb1, w2, b2, wa, wb)


def _ec(ea, w, b):
    return pl.pallas_call(
        _ec_body,
        grid=(EP // EBLK,),
        in_specs=[_rows((EBLK, EDGE_DIM)), _full((EDGE_DIM, H)), _full((1, H))],
        out_specs=_rows((EBLK, H)),
        out_shape=jax.ShapeDtypeStruct((EP, H), _F32),
    )(ea, w, b)


def _upd(s0, s1, inv, msk, h, mw2, mb2, uw1a, uw1b, ub1, uw2, ub2, g, bb):
    return pl.pallas_call(
        _upd_body,
        grid=(NP // NBLK,),
        in_specs=[_rows((NBLK, H)), _rows((NBLK, H)), _rows((NBLK, 1)),
                  _rows((NBLK, 1)), _rows((NBLK, H)),
                  _full((H, H)), _full((1, H)),
                  _full((H, 2 * H)), _full((H, 2 * H)), _full((1, 2 * H)),
                  _full((2 * H, H)), _full((1, H)),
                  _full((1, H)), _full((1, H))],
        out_specs=_rows((NBLK, H)),
        out_shape=jax.ShapeDtypeStruct((NP, H), _F32),
    )(s0, s1, inv, msk, h, mw2, mb2, uw1a, uw1b, ub1, uw2, ub2, g, bb)


def _updp(s0, s1, inv, msk, h, mw2, mb2, uw1a, uw1b, ub1, uw2, ub2, g, bb,
          wa, wb):
    return pl.pallas_call(
        _updp_body,
        grid=(NP // NBLK,),
        in_specs=[_rows((NBLK, H)), _rows((NBLK, H)), _rows((NBLK, 1)),
                  _rows((NBLK, 1)), _rows((NBLK, H)),
                  _full((H, H)), _full((1, H)),
                  _full((H, 2 * H)), _full((H, 2 * H)), _full((1, 2 * H)),
                  _full((2 * H, H)), _full((1, H)),
                  _full((1, H)), _full((1, H)),
                  _full((H, H)), _full((H, H))],
        out_specs=[_rows((NBLK, H)), _rows((NBLK, H)), _rows((NBLK, H))],
        out_shape=[jax.ShapeDtypeStruct((NP, H), _F32),
                   jax.ShapeDtypeStruct((NP, H), _F32),
                   jax.ShapeDtypeStruct((NP, H), _F32)],
    )(s0, s1, inv, msk, h, mw2, mb2, uw1a, uw1b, ub1, uw2, ub2, g, bb, wa, wb)


def _upd0p(s0, s1, c0, c1, h, mw2, mb2, uw1a, uw1b, ub1, uw2, ub2, g, bb,
           wa, wb):
    return pl.pallas_call(
        _upd0p_body,
        grid=(NP // NBLK,),
        in_specs=[_rows((NBLK, H)), _rows((NBLK, H)), _rows((NBLK, 1)),
                  _rows((NBLK, 1)), _rows((NBLK, H)),
                  _full((H, H)), _full((1, H)),
                  _full((H, 2 * H)), _full((H, 2 * H)), _full((1, 2 * H)),
                  _full((2 * H, H)), _full((1, H)),
                  _full((1, H)), _full((1, H)),
                  _full((H, H)), _full((H, H))],
        out_specs=[_rows((NBLK, H)), _rows((NBLK, 1)), _rows((NBLK, 1)),
                   _rows((NBLK, H)), _rows((NBLK, H))],
        out_shape=[jax.ShapeDtypeStruct((NP, H), _F32),
                   jax.ShapeDtypeStruct((NP, 1), _F32),
                   jax.ShapeDtypeStruct((NP, 1), _F32),
                   jax.ShapeDtypeStruct((NP, H), _F32),
                   jax.ShapeDtypeStruct((NP, H), _F32)],
    )(s0, s1, c0, c1, h, mw2, mb2, uw1a, uw1b, ub1, uw2, ub2, g, bb, wa, wb)


def _pool(h, bt):
    return pl.pallas_call(
        _pool_body,
        grid=(NP // NBLK,),
        in_specs=[_rows((NBLK, H)), _rows((NBLK, 1))],
        out_specs=_full((G, H)),
        out_shape=jax.ShapeDtypeStruct((G, H), _F32),
        scratch_shapes=[pltpu.VMEM((G, H), _F32), pltpu.VMEM((G, H), _F32)],
    )(h, bt)


def _head_a(x, w1, b1, g1, bb1, w2, b2, g2, bb2):
    return pl.pallas_call(
        _heada_body,
        in_specs=[_full((G, H)), _full((H, 2 * H)), _full((1, 2 * H)),
                  _full((1, 2 * H)), _full((1, 2 * H)),
                  _full((2 * H, 4 * H)), _full((1, 4 * H)),
                  _full((1, 4 * H)), _full((1, 4 * H))],
        out_specs=_full((G, 4 * H)),
        out_shape=jax.ShapeDtypeStruct((G, 4 * H), _F32),
    )(x, w1, b1, g1, bb1, w2, b2, g2, bb2)


def _head_b(x, w3, b3):
    cblk = 3840
    nout = 120 * 256
    return pl.pallas_call(
        _headb_body,
        grid=(nout // cblk,),
        in_specs=[_full((G, 4 * H)),
                  pl.BlockSpec((4 * H, cblk), lambda i: (0, i)),
                  pl.BlockSpec((1, cblk), lambda i: (0, i))],
        out_specs=pl.BlockSpec((G, cblk), lambda i: (0, i)),
        out_shape=jax.ShapeDtypeStruct((G, nout), _F32),
    )(x, w3, b3)


def _make_edge_kernel(with_cnt):
    mesh = plsc.VectorSubcoreMesh(core_axis_name="c", subcore_axis_name="s")
    out_type = [jax.ShapeDtypeStruct((2, SROWS, H), _F32)]
    if with_cnt:
        out_type.append(jax.ShapeDtypeStruct((2, NP), _F32))
    scratch = [
        pltpu.VMEM((2, KE), jnp.int32),
        pltpu.VMEM((KE, H), _F32),
        pltpu.VMEM((KE, H), _F32),
        pltpu.VMEM((KE, H), _F32),
        pltpu.VMEM((2, KE), jnp.int32),
        pltpu.VMEM((KE, H), _F32),
        pltpu.VMEM((KE, H), _F32),
        pltpu.VMEM((KE, H), _F32),
        pltpu.VMEM((128,), _F32),
        pltpu.VMEM_SHARED((SROWS, H), _F32),
        pltpu.VMEM_SHARED((NP,), _F32),
        pltpu.SemaphoreType.DMA,
        pltpu.SemaphoreType.DMA,
        pltpu.SemaphoreType.DMA,
        pltpu.SemaphoreType.DMA,
        pltpu.SemaphoreType.DMA,
        pltpu.SemaphoreType.DMA,
        pltpu.SemaphoreType.DMA,
        pltpu.SemaphoreType.DMA,
    ]

    def body(a_hbm, b_hbm, ec_hbm, dsp_hbm, *rest):
        if with_cnt:
            s_out, cnt_out = rest[0], rest[1]
            rest = rest[2:]
        else:
            s_out, cnt_out = rest[0], None
            rest = rest[1:]
        (ix0, ab0, bb0, eb0,
         ix1, ab1, bb1, eb1,
         onesv, s_sh, c_sh,
         semA0, semB0, semE0, semA1, semB1, semE1, semS0, semS1) = rest
        bufs = ((ix0, ab0, bb0, eb0, semA0, semB0, semE0, semS0),
                (ix1, ab1, bb1, eb1, semA1, semB1, semE1, semS1))
        c = lax.axis_index("c")
        s = lax.axis_index("s")
        wid = c * 16 + s
        r0 = jnp.minimum(s * SSLICE, SROWS - SSLICE)

        def zrow(r, _):
            for cc in range(H // 16):
                ab0[r, pl.ds(cc * 16, 16)] = jnp.zeros((16,), _F32)
            return 0
        lax.fori_loop(0, KE, zrow, 0)
        for off, sz in _SCHUNKS:
            pltpu.sync_copy(ab0.at[pl.ds(0, sz)],
                            s_sh.at[pl.ds(r0 + off, sz)])
        if with_cnt:
            def zv(j, _):
                onesv[pl.ds(j * 16, 16)] = jnp.zeros((16,), _F32)
                return 0
            lax.fori_loop(0, 8, zv, 0)
            for k in range(ROWS_PT // 128):
                pltpu.sync_copy(
                    onesv, c_sh.at[pl.ds(s * ROWS_PT + k * 128, 128)])

            def ov(j, _):
                onesv[pl.ds(j * 16, 16)] = jnp.full((16,), 1.0, _F32)
                return 0
            lax.fori_loop(0, 8, ov, 0)
        plsc.subcore_barrier()

        cbase = wid * NITER

        def prefetch(i, bset):
            ix, ab, bb, eb, sA, sB, sE, sS = bset
            ci = jnp.minimum(cbase + i, EP // KE - 1)
            pltpu.sync_copy(dsp_hbm.at[ci], ix)
            pltpu.async_copy(a_hbm.at[ix.at[0]], ab, sA)
            pltpu.async_copy(b_hbm.at[ix.at[1]], bb, sB)
            pltpu.async_copy(ec_hbm.at[pl.ds(ci * KE, KE)], eb, sE)

        def phase(i, cur, nxt):
            nix, nab, _, _, _, _, _, nsS = nxt

            @pl.when(i >= 1)
            def _():
                pltpu.make_async_copy(nab, s_sh.at[nix.at[0]], nsS).wait()
            prefetch(i + 1, nxt)
            ix, ab, bb, eb, sA, sB, sE, sS = cur
            pltpu.make_async_copy(a_hbm.at[ix.at[0]], ab, sA).wait()
            pltpu.make_async_copy(b_hbm.at[ix.at[1]], bb, sB).wait()
            pltpu.make_async_copy(ec_hbm.at[pl.ds(0, KE)], eb, sE).wait()

            @plsc.parallel_loop(0, KE, 1, unroll=2)
            def _(r2):
                for cc in range(H // 16):
                    sl = pl.ds(cc * 16, 16)
                    xv = ab[r2, sl] + bb[r2, sl] + eb[r2, sl]
                    ab[r2, sl] = xv / (1.0 + jnp.exp(-xv))

            pltpu.async_copy(ab, s_sh.at[ix.at[0]], sS, add=True)
            if with_cnt:
                pltpu.sync_copy(onesv.at[pl.ds(0, KE)], c_sh.at[ix.at[0]],
                                add=True)

        prefetch(0, bufs[0])

        def gstep(g, _):
            phase(2 * g, bufs[0], bufs[1])
            phase(2 * g + 1, bufs[1], bufs[0])
            return 0
        lax.fori_loop(0, NITER // 2, gstep, 0)
        pltpu.make_async_copy(a_hbm.at[ix0.at[0]], ab0, semA0).wait()
        pltpu.make_async_copy(b_hbm.at[ix0.at[1]], bb0, semB0).wait()
        pltpu.make_async_copy(ec_hbm.at[pl.ds(0, KE)], eb0, semE0).wait()
        pltpu.make_async_copy(ab1, s_sh.at[ix1.at[0]], semS1).wait()
        plsc.subcore_barrier()

        for off, sz in _SCHUNKS:
            rr = r0 + off
            pltpu.sync_copy(s_sh.at[pl.ds(rr, sz)], ab0.at[pl.ds(0, sz)])
            pltpu.sync_copy(ab0.at[pl.ds(0, sz)], s_out.at[c, pl.ds(rr, sz)])
        if with_cnt:
            for k in range(ROWS_PT // 128):
                rr = s * ROWS_PT + k * 128
                pltpu.sync_copy(c_sh.at[pl.ds(rr, 128)], onesv)
                pltpu.sync_copy(onesv, cnt_out.at[c, pl.ds(rr, 128)])

    return functools.partial(
        pl.kernel, mesh=mesh, out_type=out_type, scratch_types=scratch)(body)


_edge_cnt = _make_edge_kernel(True)
_edge = _make_edge_kernel(False)


def kernel(x, edge_index, edge_attr, batch, params):
    p = params
    x_p = jnp.pad(x, ((0, NP - N), (0, 0)))
    src = edge_index[0].astype(jnp.int32)
    dst = edge_index[1].astype(jnp.int32)
    pe = EP - E
    dst_p = jnp.concatenate([dst, jnp.full((pe,), N, jnp.int32)])
    src_p = jnp.concatenate([src, jnp.zeros((pe,), jnp.int32)])
    dsp = jnp.stack([dst_p.reshape(-1, KE), src_p.reshape(-1, KE)], axis=1)
    ea_p = jnp.pad(edge_attr, ((0, pe), (0, 0)))
    bt_p = jnp.concatenate(
        [batch.astype(jnp.int32), jnp.full((NP - N,), G, jnp.int32)]
    ).reshape(NP, 1)

    def r1(v):
        return v.reshape(1, -1)

    layers = p['layers']

    def msg_wa(lp):
        return lp['msg_w1'][:H]

    def msg_wb(lp):
        return lp['msg_w1'][H:2 * H]

    h, a, b = _embed(x_p, p['emb_w1'], r1(p['emb_b1']), r1(p['emb_ln_g']),
                     r1(p['emb_ln_b']), p['emb_w2'], r1(p['emb_b2']),
                     msg_wa(layers[0]), msg_wb(layers[0]))

    inv = msk = None
    for li, lp in enumerate(layers):
        wc = lp['msg_w1'][2 * H:]
        ec = _ec(ea_p, wc, r1(lp['msg_b1']))
        uw1a = lp['upd_w1'][:H]
        uw1b = lp['upd_w1'][H:]
        uargs = (lp['msg_w2'], r1(lp['msg_b2']), uw1a, uw1b,
                 r1(lp['upd_b1']), lp['upd_w2'], r1(lp['upd_b2']),
                 r1(lp['ln_g']), r1(lp['ln_b']))
        if li == 0:
            s, cnt = _edge_cnt(a, b, ec, dsp)
            s = jnp.pad(s, ((0, 0), (0, NP - SROWS), (0, 0)))
            h, inv, msk, a, b = _upd0p(
                s[0], s[1], cnt[0].reshape(NP, 1), cnt[1].reshape(NP, 1), h,
                *uargs, msg_wa(layers[1]), msg_wb(layers[1]))
        else:
            s = _edge(a, b, ec, dsp)
            if isinstance(s, (list, tuple)):
                s = s[0]
            s = jnp.pad(s, ((0, 0), (0, NP - SROWS), (0, 0)))
            if li + 1 < len(layers):
                h, a, b = _updp(s[0], s[1], inv, msk, h, *uargs,
                                msg_wa(layers[li + 1]), msg_wb(layers[li + 1]))
            else:
                h = _upd(s[0], s[1], inv, msk, h, *uargs)

    gemb = _pool(h, bt_p)
    o2 = _head_a(gemb, p['head_w1'], r1(p['head_b1']), r1(p['head_ln1_g']),
                 r1(p['head_ln1_b']), p['head_w2'], r1(p['head_b2']),
                 r1(p['head_ln2_g']), r1(p['head_ln2_b']))
    out = _head_b(o2, p['head_w3'], r1(p['head_b3']))
    return out.reshape(G, 120, 256)
